# Initial kernel scaffold; baseline (speedup 1.0000x reference)
import jax
import jax.numpy as jnp
from jax.experimental import pallas as pl


def kernel(lang, batch_input, table):
    raise NotImplementedError("write your pallas kernel here")


# speedup vs baseline: 1.8704x; 1.8704x over previous
import functools

import jax
import jax.numpy as jnp
from jax import lax
from jax.experimental import pallas as pl
from jax.experimental.pallas import tpu as pltpu
from jax.experimental.pallas import tpu_sc as plsc

BATCH = 16384
SEQ = 50
EMBED_DIM = 64

NUM_CORES = 2
NUM_SUBCORES = 16
NUM_WORKERS = NUM_CORES * NUM_SUBCORES

CHUNK = 128
NBUF = 4

TOTAL = BATCH * SEQ
NUM_CHUNKS = TOTAL // CHUNK
CHUNKS_PER_W = NUM_CHUNKS // NUM_WORKERS


def _make_sc_gather(vocab: int):
    mesh = plsc.VectorSubcoreMesh(
        core_axis_name="c", subcore_axis_name="s",
        num_cores=NUM_CORES, num_subcores=NUM_SUBCORES,
    )

    def body(idx_hbm, table_hbm, out_hbm, idx_v, rows_v, *sems):
        gsems = sems[:NBUF]
        ssems = sems[NBUF:]
        wid = lax.axis_index("s") * NUM_CORES + lax.axis_index("c")
        base = wid * CHUNKS_PER_W

        pltpu.sync_copy(idx_hbm.at[pl.ds(base, CHUNKS_PER_W)], idx_v)

        for b in range(NBUF):
            pltpu.async_copy(table_hbm.at[idx_v.at[b]], rows_v.at[b], gsems[b])

        @pl.loop(0, CHUNKS_PER_W, step=NBUF)
        def _group(g):
            for b in range(NBUF):
                pltpu.make_async_copy(
                    table_hbm.at[idx_v.at[b]], rows_v.at[b], gsems[b]
                ).wait()
                pltpu.async_copy(rows_v.at[b], out_hbm.at[base + g + b], ssems[b])
            for b in range(NBUF):
                pltpu.make_async_copy(
                    rows_v.at[b], out_hbm.at[0], ssems[b]
                ).wait()

                @pl.when(g + NBUF < CHUNKS_PER_W)
                def _refill():
                    pltpu.async_copy(
                        table_hbm.at[idx_v.at[g + NBUF + b]],
                        rows_v.at[b],
                        gsems[b],
                    )

    scratch = [
        pltpu.VMEM((CHUNKS_PER_W, CHUNK), jnp.int32),
        pltpu.VMEM((NBUF, CHUNK, EMBED_DIM), jnp.float32),
    ] + [pltpu.SemaphoreType.DMA] * (2 * NBUF)

    return pl.kernel(
        body,
        out_type=jax.ShapeDtypeStruct((NUM_CHUNKS, CHUNK, EMBED_DIM), jnp.float32),
        mesh=mesh,
        scratch_types=scratch,
        compiler_params=pltpu.CompilerParams(use_tc_tiling_on_sc=False),
    )


@jax.jit
def _lookup(batch_input, table):
    idx = batch_input.reshape(NUM_CHUNKS, CHUNK)
    out = _make_sc_gather(table.shape[0])(idx, table)
    return out.reshape(BATCH, SEQ, EMBED_DIM)


def kernel(lang, batch_input, table):
    del lang
    return _lookup(batch_input, table)


# speedup vs baseline: 1.8737x; 1.0018x over previous
import functools

import jax
import jax.numpy as jnp
from jax import lax
from jax.experimental import pallas as pl
from jax.experimental.pallas import tpu as pltpu
from jax.experimental.pallas import tpu_sc as plsc

BATCH = 16384
SEQ = 50
EMBED_DIM = 64

NUM_CORES = 2
NUM_SUBCORES = 16
NUM_WORKERS = NUM_CORES * NUM_SUBCORES

CHUNK = 128
NBUF = 8

TOTAL = BATCH * SEQ
NUM_CHUNKS = TOTAL // CHUNK
CHUNKS_PER_W = NUM_CHUNKS // NUM_WORKERS


def _make_sc_gather(vocab: int):
    mesh = plsc.VectorSubcoreMesh(
        core_axis_name="c", subcore_axis_name="s",
        num_cores=NUM_CORES, num_subcores=NUM_SUBCORES,
    )

    def body(idx_hbm, table_hbm, out_hbm, idx_v, rows_v, *sems):
        gsems = sems[:NBUF]
        ssems = sems[NBUF:]
        wid = lax.axis_index("s") * NUM_CORES + lax.axis_index("c")
        base = wid * CHUNKS_PER_W

        pltpu.sync_copy(idx_hbm.at[pl.ds(base, CHUNKS_PER_W)], idx_v)

        for b in range(NBUF):
            pltpu.async_copy(table_hbm.at[idx_v.at[b]], rows_v.at[b], gsems[b])

        @pl.loop(0, CHUNKS_PER_W, step=NBUF)
        def _group(g):
            for b in range(NBUF):
                pltpu.make_async_copy(
                    table_hbm.at[idx_v.at[b]], rows_v.at[b], gsems[b]
                ).wait()
                pltpu.async_copy(rows_v.at[b], out_hbm.at[base + g + b], ssems[b])
            for b in range(NBUF):
                pltpu.make_async_copy(
                    rows_v.at[b], out_hbm.at[0], ssems[b]
                ).wait()

                @pl.when(g + NBUF < CHUNKS_PER_W)
                def _refill():
                    pltpu.async_copy(
                        table_hbm.at[idx_v.at[g + NBUF + b]],
                        rows_v.at[b],
                        gsems[b],
                    )

    scratch = [
        pltpu.VMEM((CHUNKS_PER_W, CHUNK), jnp.int32),
        pltpu.VMEM((NBUF, CHUNK, EMBED_DIM), jnp.float32),
    ] + [pltpu.SemaphoreType.DMA] * (2 * NBUF)

    return pl.kernel(
        body,
        out_type=jax.ShapeDtypeStruct((NUM_CHUNKS, CHUNK, EMBED_DIM), jnp.float32),
        mesh=mesh,
        scratch_types=scratch,
        compiler_params=pltpu.CompilerParams(use_tc_tiling_on_sc=False),
    )


@jax.jit
def _lookup(batch_input, table):
    idx = batch_input.reshape(NUM_CHUNKS, CHUNK)
    out = _make_sc_gather(table.shape[0])(idx, table)
    return out.reshape(BATCH, SEQ, EMBED_DIM)


def kernel(lang, batch_input, table):
    del lang
    return _lookup(batch_input, table)
